# Initial kernel scaffold; baseline (speedup 1.0000x reference)
import functools

import jax
import jax.numpy as jnp
from jax import lax
from jax.experimental import pallas as pl
from jax.experimental.pallas import tpu as pltpu
from jax.experimental.pallas import tpu_sc as plsc

_N = 10000
_E = 320000
_C = 128
_NC = 2
_NS = 16
_NW = _NC * _NS
_L = 16

_B = 128
_NBLK = _E // _B
_BPW = -(-_NBLK // _NW)
_G = _B // _L


def _lane_bcast(vec, lane):
    idx = jnp.full((_L, 1), lane, jnp.int32)
    dn = lax.GatherDimensionNumbers(
        offset_dims=(), collapsed_slice_dims=(0,), start_index_map=(0,))
    return lax.gather(vec, idx, dn, slice_sizes=(1,),
                      mode=lax.GatherScatterMode.PROMISE_IN_BOUNDS)


def _tc_prep(x, w):
    def body(x_ref, w_ref, u_ref, v_ref):
        xx = x_ref[...]
        u_ref[...] = jnp.dot(xx, w_ref[0:_C, :],
                             preferred_element_type=jnp.float32)
        v_ref[...] = jnp.dot(xx, w_ref[_C:, :],
                             preferred_element_type=jnp.float32)

    return pl.pallas_call(
        body,
        out_shape=(jax.ShapeDtypeStruct((_N, _C), jnp.float32),
                   jax.ShapeDtypeStruct((_N, _C), jnp.float32)),
    )(x, w)


def _sc_attention(u, v, src, tgt, a_vec, zeros_n):
    mesh = plsc.VectorSubcoreMesh(core_axis_name="c", subcore_axis_name="s")

    @functools.partial(
        pl.kernel,
        out_type=(jax.ShapeDtypeStruct((_E,), jnp.float32),
                  jax.ShapeDtypeStruct((_NC, _N), jnp.float32)),
        mesh=mesh,
        scratch_types=[
            pltpu.VMEM((_B,), jnp.int32),
            pltpu.VMEM((_B,), jnp.int32),
            pltpu.VMEM((_B, _C), jnp.float32),
            pltpu.VMEM((_B, _C), jnp.float32),
            pltpu.VMEM((_B,), jnp.float32),
            pltpu.VMEM((_C,), jnp.float32),
            pltpu.VMEM_SHARED((_N,), jnp.float32),
        ],
    )
    def kern(u_hbm, v_hbm, src_hbm, tgt_hbm, a_hbm, z_hbm,
             exp_hbm, den_hbm, sidx, tidx, ubuf, vbuf, ebuf, abuf, den_sp):
        cid = lax.axis_index("c")
        sid = lax.axis_index("s")
        wid = cid * _NS + sid

        pltpu.sync_copy(a_hbm, abuf)

        @pl.when(sid == 0)
        def _():
            pltpu.sync_copy(z_hbm, den_sp)

        plsc.subcore_barrier()

        a_regs = [abuf[pl.ds(k * _L, _L)] for k in range(_C // _L)]
        lane = lax.iota(jnp.int32, _L)

        @pl.loop(0, _BPW)
        def _(it):
            blk = it * _NW + wid

            @pl.when(blk < _NBLK)
            def _():
                base = blk * _B
                pltpu.sync_copy(src_hbm.at[pl.ds(base, _B)], sidx)
                pltpu.sync_copy(tgt_hbm.at[pl.ds(base, _B)], tidx)
                pltpu.sync_copy(u_hbm.at[sidx], ubuf)
                pltpu.sync_copy(v_hbm.at[tidx], vbuf)

                @pl.loop(0, _G)
                def _(g):
                    evec = jnp.zeros((_L,), jnp.float32)
                    for j in range(_L):
                        r = g * _L + j
                        acc = jnp.zeros((_L,), jnp.float32)
                        for k in range(_C // _L):
                            s = (ubuf[r, pl.ds(k * _L, _L)]
                                 + vbuf[r, pl.ds(k * _L, _L)])
                            zlk = jnp.maximum(s, 0.2 * s)
                            acc = acc + zlk * a_regs[k]
                        tot = jnp.sum(acc)
                        evec = jnp.where(lane == j, tot, evec)
                    ebuf[pl.ds(g * _L, _L)] = jnp.exp(evec)

                pltpu.sync_copy(ebuf, exp_hbm.at[pl.ds(base, _B)])
                pltpu.sync_copy(ebuf, den_sp.at[sidx], add=True)

        plsc.subcore_barrier()

        @pl.when(sid == 0)
        def _():
            pltpu.sync_copy(den_sp, den_hbm.at[cid])

    return kern(u, v, src, tgt, a_vec, zeros_n)


def _sc_aggregate(x, src, tgt, exp_e, den_part, zeros_nc):
    mesh = plsc.VectorSubcoreMesh(core_axis_name="c", subcore_axis_name="s")

    @functools.partial(
        pl.kernel,
        out_type=jax.ShapeDtypeStruct((_NC, _N, _C), jnp.float32),
        mesh=mesh,
        scratch_types=[
            pltpu.VMEM((_B,), jnp.int32),
            pltpu.VMEM((_B,), jnp.int32),
            pltpu.VMEM((_B, _C), jnp.float32),
            pltpu.VMEM((_B,), jnp.float32),
            pltpu.VMEM((_N,), jnp.float32),
            pltpu.VMEM((_N,), jnp.float32),
            pltpu.VMEM_SHARED((_N, _C), jnp.float32),
        ],
    )
    def kern(x_hbm, src_hbm, tgt_hbm, exp_hbm, den_hbm, z_hbm,
             h_hbm, sidx, tidx, xbuf, ebuf, d0, d1, h_sp):
        cid = lax.axis_index("c")
        sid = lax.axis_index("s")
        wid = cid * _NS + sid

        pltpu.sync_copy(den_hbm.at[0], d0)
        pltpu.sync_copy(den_hbm.at[1], d1)

        @pl.loop(0, _N, step=_L)
        def _(i):
            d0[pl.ds(i, _L)] = d0[pl.ds(i, _L)] + d1[pl.ds(i, _L)]

        @pl.when(sid == 0)
        def _():
            pltpu.sync_copy(z_hbm, h_sp)

        plsc.subcore_barrier()

        @pl.loop(0, _BPW)
        def _(it):
            blk = it * _NW + wid

            @pl.when(blk < _NBLK)
            def _():
                base = blk * _B
                pltpu.sync_copy(src_hbm.at[pl.ds(base, _B)], sidx)
                pltpu.sync_copy(tgt_hbm.at[pl.ds(base, _B)], tidx)
                pltpu.sync_copy(exp_hbm.at[pl.ds(base, _B)], ebuf)
                pltpu.sync_copy(x_hbm.at[sidx], xbuf)

                @pl.loop(0, _G)
                def _(g):
                    srcv = sidx[pl.ds(g * _L, _L)]
                    denv = plsc.load_gather(d0, [srcv])
                    attv = ebuf[pl.ds(g * _L, _L)] / denv
                    for j in range(_L):
                        r = g * _L + j
                        sc = _lane_bcast(attv, j)
                        for k in range(_C // _L):
                            xbuf[r, pl.ds(k * _L, _L)] = (
                                xbuf[r, pl.ds(k * _L, _L)] * sc)

                pltpu.sync_copy(xbuf, h_sp.at[tidx], add=True)

        plsc.subcore_barrier()

        @pl.when(sid == 0)
        def _():
            pltpu.sync_copy(h_sp, h_hbm.at[cid])

    return kern(x, src, tgt, exp_e, den_part, zeros_nc)


def _tc_combine(h_part):
    def body(p_ref, o_ref):
        o_ref[...] = p_ref[0] + p_ref[1]

    return pl.pallas_call(
        body,
        out_shape=jax.ShapeDtypeStruct((_N, _C), jnp.float32),
    )(h_part)


def kernel(x, edge_index, w, a):
    src = edge_index[0]
    tgt = edge_index[1]
    a_vec = a.reshape(_C)
    u, v = _tc_prep(x, w)
    zeros_n = jnp.zeros((_N,), jnp.float32)
    zeros_nc = jnp.zeros((_N, _C), jnp.float32)
    exp_e, den_part = _sc_attention(u, v, src, tgt, a_vec, zeros_n)
    h_part = _sc_aggregate(x, src, tgt, exp_e, den_part, zeros_nc)
    return _tc_combine(h_part)


# speedup vs baseline: 6.0747x; 6.0747x over previous
import dataclasses
import functools

import jax
import jax.numpy as jnp
from jax import lax
from jax.experimental import pallas as pl
from jax.experimental.pallas import tpu as pltpu
from jax.experimental.pallas import tpu_sc as plsc

_N = 10000
_E = 320000
_C = 128
_NC = 2
_NS = 16
_NW = _NC * _NS
_L = 16

_SC_PARAMS = dataclasses.replace(pltpu.CompilerParams(),
                                 needs_layout_passes=False)

_B = 128
_NBLK = _E // _B
_BPW = -(-_NBLK // _NW)
_G = _B // _L


def _lane_bcast(vec, lane):
    idx = jnp.full((_L, 1), lane, jnp.int32)
    dn = lax.GatherDimensionNumbers(
        offset_dims=(), collapsed_slice_dims=(0,), start_index_map=(0,))
    return lax.gather(vec, idx, dn, slice_sizes=(1,),
                      mode=lax.GatherScatterMode.PROMISE_IN_BOUNDS)


def _tc_prep(x, w):
    def body(x_ref, w_ref, u_ref, v_ref):
        xx = x_ref[...]
        u_ref[...] = jnp.dot(xx, w_ref[0:_C, :],
                             preferred_element_type=jnp.float32)
        v_ref[...] = jnp.dot(xx, w_ref[_C:, :],
                             preferred_element_type=jnp.float32)

    return pl.pallas_call(
        body,
        out_shape=(jax.ShapeDtypeStruct((_N, _C), jnp.float32),
                   jax.ShapeDtypeStruct((_N, _C), jnp.float32)),
    )(x, w)


def _sc_attention(u, v, src, tgt, a_vec, zeros_n):
    mesh = plsc.VectorSubcoreMesh(core_axis_name="c", subcore_axis_name="s")

    @functools.partial(
        pl.kernel,
        out_type=(jax.ShapeDtypeStruct((_E,), jnp.float32),
                  jax.ShapeDtypeStruct((_NC, _N), jnp.float32)),
        mesh=mesh,
        compiler_params=_SC_PARAMS,
        scratch_types=[
            pltpu.VMEM((_B,), jnp.int32),
            pltpu.VMEM((_B,), jnp.int32),
            pltpu.VMEM((_B, _C), jnp.float32),
            pltpu.VMEM((_B, _C), jnp.float32),
            pltpu.VMEM((_B,), jnp.float32),
            pltpu.VMEM((_C,), jnp.float32),
            pltpu.VMEM_SHARED((_N,), jnp.float32),
        ],
    )
    def kern(u_hbm, v_hbm, src_hbm, tgt_hbm, a_hbm, z_hbm,
             exp_hbm, den_hbm, sidx, tidx, ubuf, vbuf, ebuf, abuf, den_sp):
        cid = lax.axis_index("c")
        sid = lax.axis_index("s")
        wid = cid * _NS + sid

        pltpu.sync_copy(a_hbm, abuf)

        @pl.when(sid == 0)
        def _():
            pltpu.sync_copy(z_hbm, den_sp)

        plsc.subcore_barrier()

        a_regs = [abuf[pl.ds(k * _L, _L)] for k in range(_C // _L)]
        lane = lax.iota(jnp.int32, _L)

        @pl.loop(0, _BPW)
        def _(it):
            blk = it * _NW + wid

            @pl.when(blk < _NBLK)
            def _():
                base = blk * _B
                pltpu.sync_copy(src_hbm.at[pl.ds(base, _B)], sidx)
                pltpu.sync_copy(tgt_hbm.at[pl.ds(base, _B)], tidx)
                pltpu.sync_copy(u_hbm.at[sidx], ubuf)
                pltpu.sync_copy(v_hbm.at[tidx], vbuf)

                @pl.loop(0, _G)
                def _(g):
                    evec = jnp.zeros((_L,), jnp.float32)
                    for j in range(_L):
                        r = g * _L + j
                        acc = jnp.zeros((_L,), jnp.float32)
                        for k in range(_C // _L):
                            s = (ubuf[r, pl.ds(k * _L, _L)]
                                 + vbuf[r, pl.ds(k * _L, _L)])
                            zlk = jnp.maximum(s, 0.2 * s)
                            acc = acc + zlk * a_regs[k]
                        tot = jnp.sum(acc)
                        evec = jnp.where(lane == j, tot, evec)
                    ebuf[pl.ds(g * _L, _L)] = jnp.exp(evec)

                pltpu.sync_copy(ebuf, exp_hbm.at[pl.ds(base, _B)])
                pltpu.sync_copy(ebuf, den_sp.at[sidx], add=True)

        plsc.subcore_barrier()

        @pl.when(sid == 0)
        def _():
            pltpu.sync_copy(den_sp, den_hbm.at[cid])

    return kern(u, v, src, tgt, a_vec, zeros_n)


def _sc_aggregate(x, src, tgt, exp_e, den_part, zeros_nc):
    mesh = plsc.VectorSubcoreMesh(core_axis_name="c", subcore_axis_name="s")

    @functools.partial(
        pl.kernel,
        out_type=jax.ShapeDtypeStruct((_NC, _N, _C), jnp.float32),
        mesh=mesh,
        compiler_params=_SC_PARAMS,
        scratch_types=[
            pltpu.VMEM((_B,), jnp.int32),
            pltpu.VMEM((_B,), jnp.int32),
            pltpu.VMEM((_B, _C), jnp.float32),
            pltpu.VMEM((_B,), jnp.float32),
            pltpu.VMEM((_N,), jnp.float32),
            pltpu.VMEM((_N,), jnp.float32),
            pltpu.VMEM_SHARED((_N, _C), jnp.float32),
        ],
    )
    def kern(x_hbm, src_hbm, tgt_hbm, exp_hbm, den_hbm, z_hbm,
             h_hbm, sidx, tidx, xbuf, ebuf, d0, d1, h_sp):
        cid = lax.axis_index("c")
        sid = lax.axis_index("s")
        wid = cid * _NS + sid

        pltpu.sync_copy(den_hbm.at[0], d0)
        pltpu.sync_copy(den_hbm.at[1], d1)

        @pl.loop(0, _N, step=_L)
        def _(i):
            d0[pl.ds(i, _L)] = d0[pl.ds(i, _L)] + d1[pl.ds(i, _L)]

        @pl.when(sid == 0)
        def _():
            pltpu.sync_copy(z_hbm, h_sp)

        plsc.subcore_barrier()

        @pl.loop(0, _BPW)
        def _(it):
            blk = it * _NW + wid

            @pl.when(blk < _NBLK)
            def _():
                base = blk * _B
                pltpu.sync_copy(src_hbm.at[pl.ds(base, _B)], sidx)
                pltpu.sync_copy(tgt_hbm.at[pl.ds(base, _B)], tidx)
                pltpu.sync_copy(exp_hbm.at[pl.ds(base, _B)], ebuf)
                pltpu.sync_copy(x_hbm.at[sidx], xbuf)

                @pl.loop(0, _G)
                def _(g):
                    srcv = sidx[pl.ds(g * _L, _L)]
                    denv = plsc.load_gather(d0, [srcv])
                    attv = ebuf[pl.ds(g * _L, _L)] / denv
                    for j in range(_L):
                        r = g * _L + j
                        sc = _lane_bcast(attv, j)
                        for k in range(_C // _L):
                            xbuf[r, pl.ds(k * _L, _L)] = (
                                xbuf[r, pl.ds(k * _L, _L)] * sc)

                pltpu.sync_copy(xbuf, h_sp.at[tidx], add=True)

        plsc.subcore_barrier()

        @pl.when(sid == 0)
        def _():
            pltpu.sync_copy(h_sp, h_hbm.at[cid])

    return kern(x, src, tgt, exp_e, den_part, zeros_nc)


def _tc_combine(h_part):
    def body(p_ref, o_ref):
        o_ref[...] = p_ref[0] + p_ref[1]

    return pl.pallas_call(
        body,
        out_shape=jax.ShapeDtypeStruct((_N, _C), jnp.float32),
    )(h_part)


def kernel(x, edge_index, w, a):
    src = edge_index[0]
    tgt = edge_index[1]
    a_vec = a.reshape(_C)
    u, v = _tc_prep(x, w)
    zeros_n = jnp.zeros((_N,), jnp.float32)
    zeros_nc = jnp.zeros((_N, _C), jnp.float32)
    exp_e, den_part = _sc_attention(u, v, src, tgt, a_vec, zeros_n)
    h_part = _sc_aggregate(x, src, tgt, exp_e, den_part, zeros_nc)
    return _tc_combine(h_part)


# speedup vs baseline: 13.1222x; 2.1601x over previous
import dataclasses
import functools

import jax
import jax.numpy as jnp
from jax import lax
from jax.experimental import pallas as pl
from jax.experimental.pallas import tpu as pltpu
from jax.experimental.pallas import tpu_sc as plsc

_N = 10000
_E = 320000
_C = 128
_NC = 2
_NS = 16
_NW = _NC * _NS
_L = 16

_SC_PARAMS = dataclasses.replace(pltpu.CompilerParams(),
                                 needs_layout_passes=False)

_B = 80
_EPT = _E // _NW
_NBW = _EPT // _B
_G = _B // _L


def _lane_bcast(vec, lane):
    idx = jnp.full((_L, 1), lane, jnp.int32)
    dn = lax.GatherDimensionNumbers(
        offset_dims=(), collapsed_slice_dims=(0,), start_index_map=(0,))
    return lax.gather(vec, idx, dn, slice_sizes=(1,),
                      mode=lax.GatherScatterMode.PROMISE_IN_BOUNDS)


def _tc_prep(x, w):
    def body(x_ref, w_ref, u_ref, v_ref):
        xx = x_ref[...]
        u_ref[...] = jnp.dot(xx, w_ref[0:_C, :],
                             preferred_element_type=jnp.float32)
        v_ref[...] = jnp.dot(xx, w_ref[_C:, :],
                             preferred_element_type=jnp.float32)

    return pl.pallas_call(
        body,
        out_shape=(jax.ShapeDtypeStruct((_N, _C), jnp.float32),
                   jax.ShapeDtypeStruct((_N, _C), jnp.float32)),
    )(x, w)


def _sc_attention(u, v, src, tgt, a_vec, zeros_n):
    mesh = plsc.VectorSubcoreMesh(core_axis_name="c", subcore_axis_name="s")

    @functools.partial(
        pl.kernel,
        out_type=(jax.ShapeDtypeStruct((_E,), jnp.float32),
                  jax.ShapeDtypeStruct((_NC, _N), jnp.float32)),
        mesh=mesh,
        compiler_params=_SC_PARAMS,
        scratch_types=[
            pltpu.VMEM((_EPT,), jnp.int32),
            pltpu.VMEM((_EPT,), jnp.int32),
            pltpu.VMEM((_B,), jnp.int32),
            pltpu.VMEM((_B,), jnp.int32),
            pltpu.VMEM((_B, _C), jnp.float32),
            pltpu.VMEM((_B, _C), jnp.float32),
            pltpu.VMEM((_B, _C), jnp.float32),
            pltpu.VMEM((_B, _C), jnp.float32),
            pltpu.VMEM((_EPT,), jnp.float32),
            pltpu.VMEM((_C,), jnp.float32),
            pltpu.VMEM_SHARED((_N,), jnp.float32),
            pltpu.SemaphoreType.DMA,
            pltpu.SemaphoreType.DMA,
            pltpu.SemaphoreType.DMA,
            pltpu.SemaphoreType.DMA,
        ],
    )
    def kern(u_hbm, v_hbm, src_hbm, tgt_hbm, a_hbm, z_hbm,
             exp_hbm, den_hbm, sidx, tidx, ssm0, ssm1, ub0, ub1, vb0, vb1,
             ebuf, abuf, den_sp, su0, su1, sv0, sv1):
        cid = lax.axis_index("c")
        sid = lax.axis_index("s")
        wid = cid * _NS + sid
        ebase = wid * _EPT

        pltpu.sync_copy(a_hbm, abuf)
        pltpu.sync_copy(src_hbm.at[pl.ds(ebase, _EPT)], sidx)
        pltpu.sync_copy(tgt_hbm.at[pl.ds(ebase, _EPT)], tidx)

        @pl.when(sid == 0)
        def _():
            pltpu.sync_copy(z_hbm, den_sp)

        plsc.subcore_barrier()

        a_regs = [abuf[pl.ds(k * _L, _L)] for k in range(_C // _L)]
        lane = lax.iota(jnp.int32, _L)

        def issue(i, ub, vb, su, sv):
            pltpu.async_copy(u_hbm.at[sidx.at[pl.ds(i * _B, _B)]], ub, su)
            pltpu.async_copy(v_hbm.at[tidx.at[pl.ds(i * _B, _B)]], vb, sv)

        def wait(i, ub, vb, su, sv):
            pltpu.make_async_copy(
                u_hbm.at[sidx.at[pl.ds(i * _B, _B)]], ub, su).wait()
            pltpu.make_async_copy(
                v_hbm.at[tidx.at[pl.ds(i * _B, _B)]], vb, sv).wait()

        def compute(i, ub, vb, ssm):
            @pl.loop(0, _G)
            def _(g):
                evec = jnp.zeros((_L,), jnp.float32)
                for j in range(_L):
                    r = g * _L + j
                    acc0 = jnp.zeros((_L,), jnp.float32)
                    acc1 = jnp.zeros((_L,), jnp.float32)
                    for k in range(0, _C // _L, 2):
                        s0 = ub[r, pl.ds(k * _L, _L)] + vb[r, pl.ds(k * _L, _L)]
                        acc0 = acc0 + jnp.maximum(s0, 0.2 * s0) * a_regs[k]
                        s1 = (ub[r, pl.ds((k + 1) * _L, _L)]
                              + vb[r, pl.ds((k + 1) * _L, _L)])
                        acc1 = acc1 + jnp.maximum(s1, 0.2 * s1) * a_regs[k + 1]
                    tot = jnp.sum(acc0 + acc1)
                    evec = jnp.where(lane == j, tot, evec)
                ebuf[pl.ds(i * _B + g * _L, _L)] = jnp.exp(evec)
                ssm[pl.ds(g * _L, _L)] = sidx[pl.ds(i * _B + g * _L, _L)]
            pltpu.sync_copy(ebuf.at[pl.ds(i * _B, _B)],
                            den_sp.at[ssm], add=True)

        issue(0, ub0, vb0, su0, sv0)

        @pl.loop(0, _NBW + 1, step=2)
        def _(it):
            @pl.when(it + 1 < _NBW)
            def _():
                issue(it + 1, ub1, vb1, su1, sv1)
            wait(it, ub0, vb0, su0, sv0)
            compute(it, ub0, vb0, ssm0)

            @pl.when(it + 2 < _NBW)
            def _():
                issue(it + 2, ub0, vb0, su0, sv0)

            @pl.when(it + 1 < _NBW)
            def _():
                wait(it + 1, ub1, vb1, su1, sv1)
                compute(it + 1, ub1, vb1, ssm1)

        pltpu.sync_copy(ebuf, exp_hbm.at[pl.ds(ebase, _EPT)])
        plsc.subcore_barrier()

        @pl.when(sid == 0)
        def _():
            pltpu.sync_copy(den_sp, den_hbm.at[cid])

    return kern(u, v, src, tgt, a_vec, zeros_n)


def _tc_densum(den_part):
    def body(p_ref, o_ref):
        o_ref[...] = p_ref[0:1, :] + p_ref[1:2, :]

    return pl.pallas_call(
        body,
        out_shape=jax.ShapeDtypeStruct((1, _N), jnp.float32),
    )(den_part)


def _sc_aggregate(x, src, tgt, exp_e, den_sum, zeros_nc):
    mesh = plsc.VectorSubcoreMesh(core_axis_name="c", subcore_axis_name="s")

    @functools.partial(
        pl.kernel,
        out_type=jax.ShapeDtypeStruct((_NC, _N, _C), jnp.float32),
        mesh=mesh,
        compiler_params=_SC_PARAMS,
        scratch_types=[
            pltpu.VMEM((_B,), jnp.int32),
            pltpu.VMEM((_B,), jnp.int32),
            pltpu.VMEM((_B,), jnp.int32),
            pltpu.VMEM((_B,), jnp.int32),
            pltpu.VMEM((_B,), jnp.float32),
            pltpu.VMEM((_B,), jnp.float32),
            pltpu.VMEM((_B, _C), jnp.float32),
            pltpu.VMEM((_B, _C), jnp.float32),
            pltpu.VMEM((_N,), jnp.float32),
            pltpu.VMEM_SHARED((_N, _C), jnp.float32),
            pltpu.SemaphoreType.DMA,
            pltpu.SemaphoreType.DMA,
            pltpu.SemaphoreType.DMA,
            pltpu.SemaphoreType.DMA,
            pltpu.SemaphoreType.DMA,
            pltpu.SemaphoreType.DMA,
            pltpu.SemaphoreType.DMA,
            pltpu.SemaphoreType.DMA,
        ],
    )
    def kern(x_hbm, src_hbm, tgt_hbm, exp_hbm, den_hbm, z_hbm,
             h_hbm, si0, si1, ti0, ti1, eb0, eb1, xb0, xb1, d0, h_sp,
             ss0, ss1, st0, st1, se0, se1, sx0, sx1):
        cid = lax.axis_index("c")
        sid = lax.axis_index("s")
        wid = cid * _NS + sid
        ebase = wid * _EPT

        pltpu.sync_copy(den_hbm, d0)

        @pl.when(sid == 0)
        def _():
            pltpu.sync_copy(z_hbm, h_sp)

        plsc.subcore_barrier()

        def issue_idx(i, si, ti, eb, ss, st, se):
            pltpu.async_copy(src_hbm.at[pl.ds(ebase + i * _B, _B)], si, ss)
            pltpu.async_copy(tgt_hbm.at[pl.ds(ebase + i * _B, _B)], ti, st)
            pltpu.async_copy(exp_hbm.at[pl.ds(ebase + i * _B, _B)], eb, se)

        def wait_idx(i, si, ti, eb, ss, st, se):
            pltpu.make_async_copy(
                src_hbm.at[pl.ds(ebase + i * _B, _B)], si, ss).wait()
            pltpu.make_async_copy(
                tgt_hbm.at[pl.ds(ebase + i * _B, _B)], ti, st).wait()
            pltpu.make_async_copy(
                exp_hbm.at[pl.ds(ebase + i * _B, _B)], eb, se).wait()

        def issue_x(si, xb, sx):
            pltpu.async_copy(x_hbm.at[si], xb, sx)

        def wait_x(si, xb, sx):
            pltpu.make_async_copy(x_hbm.at[si], xb, sx).wait()

        def compute(si, ti, eb, xb):
            @pl.loop(0, _G)
            def _(g):
                srcv = si[pl.ds(g * _L, _L)]
                denv = plsc.load_gather(d0, [srcv])
                attv = eb[pl.ds(g * _L, _L)] / denv
                for j in range(_L):
                    r = g * _L + j
                    sc = _lane_bcast(attv, j)
                    for k in range(_C // _L):
                        xb[r, pl.ds(k * _L, _L)] = xb[r, pl.ds(k * _L, _L)] * sc
            pltpu.sync_copy(xb, h_sp.at[ti], add=True)

        pltpu.sync_copy(src_hbm.at[pl.ds(ebase, _B)], si0)
        pltpu.sync_copy(tgt_hbm.at[pl.ds(ebase, _B)], ti0)
        pltpu.sync_copy(exp_hbm.at[pl.ds(ebase, _B)], eb0)
        issue_x(si0, xb0, sx0)
        issue_idx(1, si1, ti1, eb1, ss1, st1, se1)

        @pl.loop(0, _NBW + 1, step=2)
        def _(it):
            @pl.when(it + 1 < _NBW)
            def _():
                wait_idx(it + 1, si1, ti1, eb1, ss1, st1, se1)
                issue_x(si1, xb1, sx1)
            wait_x(si0, xb0, sx0)
            compute(si0, ti0, eb0, xb0)

            @pl.when(it + 2 < _NBW)
            def _():
                issue_idx(it + 2, si0, ti0, eb0, ss0, st0, se0)

            @pl.when(it + 1 < _NBW)
            def _():
                @pl.when(it + 2 < _NBW)
                def _():
                    wait_idx(it + 2, si0, ti0, eb0, ss0, st0, se0)
                    issue_x(si0, xb0, sx0)
                wait_x(si1, xb1, sx1)
                compute(si1, ti1, eb1, xb1)

                @pl.when(it + 3 < _NBW)
                def _():
                    issue_idx(it + 3, si1, ti1, eb1, ss1, st1, se1)

        plsc.subcore_barrier()

        @pl.when(sid == 0)
        def _():
            pltpu.sync_copy(h_sp, h_hbm.at[cid])

    return kern(x, src, tgt, exp_e, den_sum, zeros_nc)


def _tc_combine(h_part):
    def body(p_ref, o_ref):
        o_ref[...] = p_ref[0] + p_ref[1]

    return pl.pallas_call(
        body,
        out_shape=jax.ShapeDtypeStruct((_N, _C), jnp.float32),
    )(h_part)


def kernel(x, edge_index, w, a):
    src = edge_index[0]
    tgt = edge_index[1]
    a_vec = a.reshape(_C)
    u, v = _tc_prep(x, w)
    zeros_n = jnp.zeros((_N,), jnp.float32)
    zeros_nc = jnp.zeros((_N, _C), jnp.float32)
    exp_e, den_part = _sc_attention(u, v, src, tgt, a_vec, zeros_n)
    den_sum = _tc_densum(den_part).reshape(_N)
    h_part = _sc_aggregate(x, src, tgt, exp_e, den_sum, zeros_nc)
    return _tc_combine(h_part)
